# baseline (speedup 1.0000x reference)
import functools

import jax
import jax.numpy as jnp
from jax import lax
from jax.experimental import pallas as pl
from jax.experimental.pallas import tpu as pltpu
from jax.experimental.pallas import tpu_sc as plsc

NC = 2
NS = 16
NW = NC * NS
LANES = 16


def _make_sc_kernel(B, D, b_per_w):
    n_chunks = b_per_w // LANES
    mesh = plsc.VectorSubcoreMesh(core_axis_name="c", subcore_axis_name="s")

    @functools.partial(
        pl.kernel,
        mesh=mesh,
        out_type=jax.ShapeDtypeStruct((NW, LANES), jnp.float32),
        compiler_params=pltpu.CompilerParams(use_tc_tiling_on_sc=True),
        scratch_types=[
            pltpu.VMEM((b_per_w,), jnp.int32),
            pltpu.VMEM((b_per_w, D), jnp.float32),
            pltpu.VMEM((LANES, 8, D), jnp.float32),
            pltpu.VMEM((LANES,), jnp.float32),
            pltpu.SemaphoreType.DMA,
            pltpu.SemaphoreType.DMA((LANES,)),
        ],
    )
    def sc_kernel(emb_hbm, idx_hbm, tbl_hbm, out_hbm,
                  idx_v, emb_v, blk_v, res_v, sem_e, sem_g):
        wid = lax.axis_index("s") * NC + lax.axis_index("c")
        base = wid * b_per_w

        emb_cp = pltpu.async_copy(emb_hbm.at[pl.ds(base, b_per_w)], emb_v, sem_e)
        pltpu.sync_copy(idx_hbm.at[pl.ds(base, b_per_w)], idx_v)

        def fire(l, slot):
            blk = pl.multiple_of((l >> 3) << 3, 8)
            pltpu.async_copy(tbl_hbm.at[pl.ds(blk, 8)], blk_v.at[slot],
                             sem_g.at[slot])

        v0 = idx_v[pl.ds(0, LANES)]
        for j in range(LANES):
            fire(v0[j], j)
        emb_cp.wait()

        def body(g, accs):
            out = list(accs)
            vc = idx_v[pl.ds(g * LANES, LANES)]
            gn = jnp.minimum(g + 1, n_chunks - 1)
            vn = idx_v[pl.ds(gn * LANES, LANES)]
            not_last = g + 1 < n_chunks
            for j in range(LANES):
                l = vc[j]
                sub = l & 7
                r = g * LANES + j
                pltpu.make_async_copy(
                    tbl_hbm.at[pl.ds(0, 8)], blk_v.at[j], sem_g.at[j]).wait()
                for f in range(D // LANES):
                    sl = pl.ds(f * LANES, LANES)
                    d = emb_v[r, sl] - blk_v[j, sub, sl]
                    out[f] = out[f] + d * d
                ln = vn[j]

                @pl.when(not_last)
                def _():
                    fire(ln, j)
            return tuple(out)

        zero = jnp.zeros((LANES,), jnp.float32)
        accs = lax.fori_loop(0, n_chunks, body, (zero,) * (D // LANES))
        total = accs[0]
        for a in accs[1:]:
            total = total + a
        res_v[...] = total
        pltpu.sync_copy(res_v, out_hbm.at[wid])

    return sc_kernel


def kernel(embedding_batch, label_batch, class_centers):
    B, D = embedding_batch.shape
    sc_kernel = _make_sc_kernel(B, D, B // NW)
    partials = sc_kernel(embedding_batch,
                         label_batch.astype(jnp.int32),
                         class_centers)
    return jnp.sum(partials) / B


# speedup vs baseline: 1.6975x; 1.6975x over previous
import functools

import jax
import jax.numpy as jnp
from jax import lax
from jax.experimental import pallas as pl
from jax.experimental.pallas import tpu as pltpu
from jax.experimental.pallas import tpu_sc as plsc

NC = 2
NS = 16
NW = NC * NS
LANES = 16
NBUF = 4


def _make_sc_kernel(B, D, b_per_w):
    n_chunks = b_per_w // LANES
    mesh = plsc.VectorSubcoreMesh(core_axis_name="c", subcore_axis_name="s")

    @functools.partial(
        pl.kernel,
        mesh=mesh,
        out_type=jax.ShapeDtypeStruct((NW, LANES), jnp.float32),
        compiler_params=pltpu.CompilerParams(use_tc_tiling_on_sc=True,
                                             needs_layout_passes=False),
        scratch_types=[
            pltpu.VMEM((b_per_w,), jnp.int32),
            pltpu.VMEM((b_per_w, D), jnp.float32),
            pltpu.VMEM((NBUF, D, 128), jnp.float32),
            pltpu.VMEM((LANES,), jnp.float32),
            pltpu.SemaphoreType.DMA,
            pltpu.SemaphoreType.DMA((NBUF,)),
        ],
    )
    def sc_kernel(emb_hbm, idx_hbm, tblT_hbm, out_hbm,
                  idx_v, emb_v, blk_v, res_v, sem_e, sem_g):
        wid = lax.axis_index("s") * NC + lax.axis_index("c")
        base = wid * b_per_w

        emb_cp = pltpu.async_copy(emb_hbm.at[pl.ds(base, b_per_w)], emb_v, sem_e)
        pltpu.sync_copy(idx_hbm.at[pl.ds(base, b_per_w)], idx_v)

        def fire(l, slot):
            col = pl.multiple_of((l >> 7) << 7, 128)
            pltpu.async_copy(tblT_hbm.at[:, pl.ds(col, 128)], blk_v.at[slot],
                             sem_g.at[slot])

        v0 = idx_v[pl.ds(0, LANES)]
        for j in range(NBUF):
            fire(v0[j], j)
        emb_cp.wait()

        lane = lax.iota(jnp.int32, LANES)

        def body(g, accs):
            out = list(accs)
            vc = idx_v[pl.ds(g * LANES, LANES)]
            gn = jnp.minimum(g + 1, n_chunks - 1)
            vn = idx_v[pl.ds(gn * LANES, LANES)]
            not_last = g + 1 < n_chunks
            for h in range(LANES // NBUF):
                for j in range(NBUF):
                    l = vc[NBUF * h + j]
                    sub = jnp.full((LANES,), l & 127, jnp.int32)
                    r = g * LANES + NBUF * h + j
                    pltpu.make_async_copy(
                        tblT_hbm.at[:, pl.ds(0, 128)], blk_v.at[j],
                        sem_g.at[j]).wait()
                    for f in range(D // LANES):
                        sl = pl.ds(f * LANES, LANES)
                        c = plsc.load_gather(
                            blk_v.at[j], [lane + f * LANES, sub])
                        d = emb_v[r, sl] - c
                        out[f] = out[f] + d * d
                    nxt = NBUF * (h + 1) + j
                    if nxt < LANES:
                        fire(vc[nxt], j)
                    else:
                        ln = vn[j]

                        @pl.when(not_last)
                        def _():
                            fire(ln, j)
            return tuple(out)

        zero = jnp.zeros((LANES,), jnp.float32)
        accs = lax.fori_loop(0, n_chunks, body, (zero,) * (D // LANES))
        total = accs[0]
        for a in accs[1:]:
            total = total + a
        res_v[...] = total
        pltpu.sync_copy(res_v, out_hbm.at[wid])

    return sc_kernel


def kernel(embedding_batch, label_batch, class_centers):
    B, D = embedding_batch.shape
    sc_kernel = _make_sc_kernel(B, D, B // NW)
    partials = sc_kernel(embedding_batch,
                         label_batch.astype(jnp.int32),
                         class_centers.T)
    return jnp.sum(partials) / B
